# baseline (speedup 1.0000x reference)
import functools

import jax
import jax.numpy as jnp
from jax import lax
from jax.experimental import pallas as pl
from jax.experimental.pallas import tpu as pltpu
from jax.experimental.pallas import tpu_sc as plsc

BATCH = 16384
HIST = 50
HIDDEN = 64
TOTAL = BATCH * HIST
LINES = TOTAL // 2

NUM_CORES = 2
NUM_SUBCORES = 16
NUM_WORKERS = NUM_CORES * NUM_SUBCORES

CHUNK = 128
K = 2
LINES_PER_BUF = CHUNK * K
LINES_PER_WORKER = LINES // NUM_WORKERS
STEPS = LINES_PER_WORKER // CHUNK
NCHUNK = LINES_PER_WORKER // LINES_PER_BUF
GROUPS = NCHUNK // 2


def _build_kernel():
    mesh = plsc.VectorSubcoreMesh(core_axis_name="c", subcore_axis_name="s")

    @functools.partial(
        pl.kernel,
        mesh=mesh,
        compiler_params=pltpu.CompilerParams(use_tc_tiling_on_sc=False),
        out_type=jax.ShapeDtypeStruct((LINES, 2 * HIDDEN), jnp.float32),
        scratch_types=[
            pltpu.VMEM((STEPS, CHUNK), jnp.int32),
            pltpu.VMEM((STEPS, CHUNK), jnp.int32),
            pltpu.VMEM((LINES_PER_BUF, HIDDEN), jnp.float32),
            pltpu.VMEM((LINES_PER_BUF, HIDDEN), jnp.float32),
            pltpu.VMEM((LINES_PER_BUF, HIDDEN), jnp.float32),
            pltpu.VMEM((LINES_PER_BUF, HIDDEN), jnp.float32),
            pltpu.SemaphoreType.DMA,
            pltpu.SemaphoreType.DMA,
        ],
    )
    def emb_kernel(
        idx_e_hbm, idx_o_hbm, table_hbm, out_hbm, idx_e, idx_o,
        buf_ae, buf_ao, buf_be, buf_bo, sem_a, sem_b
    ):
        wid = lax.axis_index("s") * NUM_CORES + lax.axis_index("c")
        base_line = wid * LINES_PER_WORKER
        pltpu.sync_copy(idx_e_hbm.at[pl.ds(wid * STEPS, STEPS)], idx_e)
        pltpu.sync_copy(idx_o_hbm.at[pl.ds(wid * STEPS, STEPS)], idx_o)

        def fire(c, bufe, bufo, sem):
            for k in range(K):
                j = c * K + k
                dst = pl.ds(k * CHUNK, CHUNK)
                pltpu.async_copy(table_hbm.at[idx_e.at[j]], bufe.at[dst], sem)
                pltpu.async_copy(table_hbm.at[idx_o.at[j]], bufo.at[dst], sem)

        def drain_and_write(c, bufe, bufo, sem):
            for k in range(2 * K):
                pltpu.make_async_copy(
                    table_hbm.at[pl.ds(0, CHUNK)],
                    bufe.at[pl.ds(0, CHUNK)],
                    sem,
                ).wait()
            lines = pl.ds(base_line + c * LINES_PER_BUF, LINES_PER_BUF)
            pltpu.sync_copy(bufe, out_hbm.at[lines, pl.ds(0, HIDDEN)])
            pltpu.sync_copy(bufo, out_hbm.at[lines, pl.ds(HIDDEN, HIDDEN)])

        fire(0, buf_ae, buf_ao, sem_a)

        def group(g, carry):
            for p, (bufe, bufo, sem, obufe, obufo, osem) in enumerate(
                (
                    (buf_ae, buf_ao, sem_a, buf_be, buf_bo, sem_b),
                    (buf_be, buf_bo, sem_b, buf_ae, buf_ao, sem_a),
                )
            ):
                c = 2 * g + p

                @pl.when(c + 1 < NCHUNK)
                def _():
                    fire(c + 1, obufe, obufo, osem)

                drain_and_write(c, bufe, bufo, sem)
            return carry

        lax.fori_loop(0, GROUPS, group, 0)

    return emb_kernel


_EMB_KERNEL = _build_kernel()


@jax.jit
def kernel(input_ids, weight):
    idx = input_ids.reshape(LINES, 2).astype(jnp.int32)
    idx_e = idx[:, 0].reshape(LINES // CHUNK, CHUNK)
    idx_o = idx[:, 1].reshape(LINES // CHUNK, CHUNK)
    out = _EMB_KERNEL(idx_e, idx_o, weight)
    return out.reshape(BATCH, HIST, HIDDEN)


# speedup vs baseline: 1.0600x; 1.0600x over previous
import functools

import jax
import jax.numpy as jnp
from jax import lax
from jax.experimental import pallas as pl
from jax.experimental.pallas import tpu as pltpu
from jax.experimental.pallas import tpu_sc as plsc

BATCH = 16384
HIST = 50
HIDDEN = 64
HHALF = HIST // 2

NUM_CORES = 2
NUM_SUBCORES = 16
NUM_WORKERS = NUM_CORES * NUM_SUBCORES

B_PER_WORKER = BATCH // NUM_WORKERS
NBB = 8
NCHUNK = B_PER_WORKER // NBB
GROUPS = NCHUNK // 2


def _build_kernel():
    mesh = plsc.VectorSubcoreMesh(core_axis_name="c", subcore_axis_name="s")

    half = jax.ShapeDtypeStruct((BATCH, HHALF, HIDDEN), jnp.float32)

    @functools.partial(
        pl.kernel,
        mesh=mesh,
        compiler_params=pltpu.CompilerParams(use_tc_tiling_on_sc=False),
        out_type=(half, half),
        scratch_types=[
            pltpu.VMEM((B_PER_WORKER, HIST), jnp.int32),
            pltpu.VMEM((NBB, HIST, HIDDEN), jnp.float32),
            pltpu.VMEM((NBB, HIST, HIDDEN), jnp.float32),
            pltpu.SemaphoreType.DMA,
            pltpu.SemaphoreType.DMA,
        ],
    )
    def emb_kernel(
        idx_hbm, table_hbm, out1_hbm, out2_hbm, idx_v, buf_a, buf_b, sem_a, sem_b
    ):
        wid = lax.axis_index("s") * NUM_CORES + lax.axis_index("c")
        base_b = wid * B_PER_WORKER
        pltpu.sync_copy(idx_hbm.at[pl.ds(base_b, B_PER_WORKER)], idx_v)

        def fire(c, buf, sem):
            for k in range(NBB):
                pltpu.async_copy(
                    table_hbm.at[idx_v.at[c * NBB + k]],
                    buf.at[k],
                    sem,
                )

        def drain_and_write(c, buf, sem):
            for k in range(NBB):
                pltpu.make_async_copy(
                    table_hbm.at[pl.ds(0, HIST)],
                    buf.at[k],
                    sem,
                ).wait()
            bs = pl.ds(base_b + c * NBB, NBB)
            pltpu.sync_copy(buf.at[:, pl.ds(0, HHALF)], out1_hbm.at[bs])
            pltpu.sync_copy(buf.at[:, pl.ds(HHALF, HHALF)], out2_hbm.at[bs])

        fire(0, buf_a, sem_a)

        def group(g, carry):
            for p, (buf, sem, obuf, osem) in enumerate(
                ((buf_a, sem_a, buf_b, sem_b), (buf_b, sem_b, buf_a, sem_a))
            ):
                c = 2 * g + p

                @pl.when(c + 1 < NCHUNK)
                def _():
                    fire(c + 1, obuf, osem)

                drain_and_write(c, buf, sem)
            return carry

        lax.fori_loop(0, GROUPS, group, 0)

    return emb_kernel


_EMB_KERNEL = _build_kernel()


@jax.jit
def kernel(input_ids, weight):
    o1, o2 = _EMB_KERNEL(input_ids.astype(jnp.int32), weight)
    return jnp.concatenate([o1, o2], axis=1)
